# baseline (speedup 1.0000x reference)
import jax
import jax.numpy as jnp
from jax import lax
from jax.experimental import pallas as pl
from jax.experimental.pallas import tpu as pltpu
from jax.experimental.pallas import tpu_sc as plsc

MAXLEN = 2048
VOCAB = 100000
D_MODEL = 768
BATCH = 4

NUM_CORES = 2
NUM_SUBCORES = 16
NW = NUM_CORES * NUM_SUBCORES
ROWS = BATCH * MAXLEN
P_PER_W = MAXLEN // NW
SCHUNK = 16
NSC = P_PER_W // SCHUNK
LANES = 16
D_VECS = D_MODEL // LANES


def _emb_kernel(x_hbm, table_hbm, pos_hbm, out_hbm,
                idx_all, pbuf0, pbuf1,
                b00, b01, b02, b03, b10, b11, b12, b13,
                gsem0, gsem1, psem0, psem1, ssem0, ssem1):
    wid = lax.axis_index("s") * NUM_CORES + lax.axis_index("c")
    pos_lo = wid * P_PER_W

    bufsets = ((b00, b01, b02, b03), (b10, b11, b12, b13))
    pbufs = (pbuf0, pbuf1)
    gsems = (gsem0, gsem1)
    psems = (psem0, psem1)
    ssems = (ssem0, ssem1)

    idx_cps = [
        pltpu.async_copy(x_hbm.at[pl.ds(b * MAXLEN + pos_lo, P_PER_W)],
                         idx_all.at[pl.ds(b * P_PER_W, P_PER_W)], gsem0)
        for b in range(BATCH)
    ]
    for cp in idx_cps:
        cp.wait()

    def issue(s):
        k = s & 1
        gs = [
            pltpu.async_copy(
                table_hbm.at[idx_all.at[pl.ds(b * P_PER_W + s * SCHUNK, SCHUNK)]],
                bufsets[k][b], gsems[k])
            for b in range(BATCH)
        ]
        pcp = pltpu.async_copy(
            pos_hbm.at[pl.ds(pos_lo + s * SCHUNK, SCHUNK), :], pbufs[k], psems[k])
        return gs + [pcp]

    pend = {0: issue(0)}
    stores = {}
    for s in range(NSC):
        k = s & 1
        if s + 1 < NSC:
            for st in stores.pop(s - 1, ()):
                st.wait()
            pend[s + 1] = issue(s + 1)
        for cp in pend.pop(s):
            cp.wait()

        bufs = bufsets[k]
        pbuf = pbufs[k]

        def add_row(r, _, bufs=bufs, pbuf=pbuf):
            for j in range(D_VECS):
                sl = pl.ds(j * LANES, LANES)
                pv = pbuf[r, sl]
                for b in range(BATCH):
                    bufs[b][r, sl] = bufs[b][r, sl] + pv
            return 0

        lax.fori_loop(0, SCHUNK, add_row, 0)
        stores[s] = [
            pltpu.async_copy(
                bufs[b],
                out_hbm.at[pl.ds(b * MAXLEN + pos_lo + s * SCHUNK, SCHUNK), :],
                ssems[k])
            for b in range(BATCH)
        ]
    for sts in stores.values():
        for st in sts:
            st.wait()


def kernel(x, table, pos_enc):
    flat_x = x.reshape(ROWS)
    mesh = plsc.VectorSubcoreMesh(core_axis_name="c", subcore_axis_name="s")
    run = pl.kernel(
        _emb_kernel,
        out_type=jax.ShapeDtypeStruct((ROWS, D_MODEL), jnp.float32),
        mesh=mesh,
        scratch_types=[
            pltpu.VMEM((BATCH * P_PER_W,), jnp.int32),
            pltpu.VMEM((SCHUNK, D_MODEL), jnp.float32),
            pltpu.VMEM((SCHUNK, D_MODEL), jnp.float32),
        ] + [pltpu.VMEM((SCHUNK, D_MODEL), jnp.float32) for _ in range(8)] + [
            pltpu.SemaphoreType.DMA,
            pltpu.SemaphoreType.DMA,
            pltpu.SemaphoreType.DMA,
            pltpu.SemaphoreType.DMA,
            pltpu.SemaphoreType.DMA,
            pltpu.SemaphoreType.DMA,
        ],
    )
    out = run(flat_x, table, pos_enc)
    return out.reshape(BATCH, MAXLEN, D_MODEL)
